# baseline (speedup 1.0000x reference)
import jax
import jax.numpy as jnp
from jax import lax
from jax.experimental import pallas as pl
from jax.experimental.pallas import tpu as pltpu

N = 16384
M = 256
K = 8
DIM = 16
JITTER = 1e-4
LENGTHSCALE = 1.0
VARIANCE = 1.0
Q = 256
NB = 128
SB = 32

_F32 = jnp.float32


def _prep_body(z_ref, lur_ref, lu_ref, l_ref, s_ref, zt_ref, z2_ref,
               lt_ref, spnl_ref):
    Z = z_ref[:]
    raw = lur_ref[:]
    r = lax.broadcasted_iota(jnp.int32, (M, M), 0)
    c = lax.broadcasted_iota(jnp.int32, (M, M), 1)
    eye = (r == c).astype(_F32)
    lu = jnp.where(c < r, raw, 0.0) + jnp.where(c == r, jnp.exp(raw), 0.0)
    lu_ref[:] = lu
    s_ref[:] = lax.dot_general(lu, lu, (((1,), (1,)), ((), ())),
                               preferred_element_type=_F32)
    zt_ref[:] = lax.dot_general(Z, eye, (((0,), (0,)), ((), ())),
                                preferred_element_type=_F32)
    z2c = jnp.sum(Z * Z, axis=1, keepdims=True)
    z2r = lax.dot_general(jnp.ones((1, DIM), _F32), Z * Z,
                          (((1,), (1,)), ((), ())),
                          preferred_element_type=_F32)
    z2_ref[:] = z2r
    zz = lax.dot_general(Z, Z, (((1,), (1,)), ((), ())),
                         preferred_element_type=_F32)
    d2 = jnp.maximum(z2c + z2r - 2.0 * zz, 0.0)
    kzz = VARIANCE * jnp.exp(-0.5 * d2 / (LENGTHSCALE ** 2))
    aw0 = kzz + JITTER * eye

    c_m = lax.broadcasted_iota(jnp.int32, (1, M), 1)
    lt_ref[:] = jnp.zeros((M, M), _F32)

    def make_body(s):
        def body(j, carry):
            rowj = spnl_ref[pl.ds(j, 1), :]
            jj = s + j
            ehot = (c_m == jj).astype(_F32)
            piv = jnp.sum(rowj * ehot)
            d = jnp.sqrt(piv)
            u = jnp.where(c_m > jj, rowj, 0.0) / d
            lt_ref[pl.ds(jj, 1), :] = u + d * ehot
            outer = lax.dot_general(u[:, s:s + SB], u,
                                    (((0,), (0,)), ((), ())),
                                    preferred_element_type=_F32)
            spnl_ref[:] = spnl_ref[:] - outer
            return carry
        return body

    for s in range(0, M, SB):
        blk = aw0[s:s + SB, :]
        if s:
            lts = lt_ref[:, s:s + SB]
            blk = blk - lax.dot_general(lts, lt_ref[:],
                                        (((0,), (0,)), ((), ())),
                                        preferred_element_type=_F32)
        spnl_ref[:] = blk
        lax.fori_loop(0, SB, make_body(s), 0)

    l_ref[:] = lax.dot_general(eye, lt_ref[:], (((1,), (1,)), ((), ())),
                               preferred_element_type=_F32)


def _main_body(x_ref, zt_ref, z2_ref, s_ref, mu_ref, mean_ref, scale_ref):
    x = x_ref[:]
    zt = zt_ref[:]
    z2 = z2_ref[:]
    x2 = jnp.sum(x * x, axis=1, keepdims=True)
    xz = lax.dot_general(x, zt, (((1,), (0,)), ((), ())),
                         preferred_element_type=_F32)
    d2 = jnp.maximum(x2 + z2 - 2.0 * xz, 0.0)
    lane_f = lax.broadcasted_iota(jnp.int32, (Q, M), 1).astype(_F32)
    idxs, vals = [], []
    d2w = d2
    for _ in range(K):
        mval = jnp.min(d2w, axis=1, keepdims=True)
        cand = jnp.where(d2w == mval, lane_f, _F32(1e9))
        am = jnp.min(cand, axis=1, keepdims=True)
        idxs.append(am)
        vals.append(mval)
        d2w = jnp.where(lane_f == am, _F32(3.4e38), d2w)
    idxf = jnp.concatenate(idxs, axis=1)
    d2k = jnp.concatenate(vals, axis=1)

    r_q = lax.broadcasted_iota(jnp.int32, (Q, Q), 0)
    c_q = lax.broadcasted_iota(jnp.int32, (Q, Q), 1)
    eye_q = (r_q == c_q).astype(_F32)
    idx_t = lax.dot_general(idxf, eye_q, (((0,), (0,)), ((), ())),
                            preferred_element_type=_F32)
    d2k_t = lax.dot_general(d2k, eye_q, (((0,), (0,)), ((), ())),
                            preferred_element_type=_F32)
    kxz_t = VARIANCE * jnp.exp(-0.5 * d2k_t / (LENGTHSCALE ** 2))

    r_mq = lax.broadcasted_iota(jnp.int32, (M, Q), 0).astype(_F32)
    ph = [(r_mq == idx_t[j:j + 1, :]).astype(_F32) for j in range(K)]
    zg = [lax.dot_general(zt, ph[j], (((1,), (0,)), ((), ())),
                          preferred_element_type=_F32) for j in range(K)]
    z2k = [jnp.sum(zg[j] * zg[j], axis=0, keepdims=True) for j in range(K)]

    gpair = {}
    for i in range(K):
        for j in range(i, K):
            gpair[(i, j)] = jnp.sum(zg[i] * zg[j], axis=0, keepdims=True)
    rows = []
    for i in range(K):
        for j in range(K):
            gij = gpair[(i, j)] if i <= j else gpair[(j, i)]
            rows.append(z2k[i] + z2k[j] - 2.0 * gij)
    d2z = jnp.maximum(jnp.concatenate(rows, axis=0), 0.0)
    ks = VARIANCE * jnp.exp(-0.5 * d2z / (LENGTHSCALE ** 2))
    r_kk1 = lax.broadcasted_iota(jnp.int32, (K, K, 1), 0)
    c_kk1 = lax.broadcasted_iota(jnp.int32, (K, K, 1), 1)
    b = ks.reshape(K, K, Q) + 2.0 * JITTER * (r_kk1 == c_kk1).astype(_F32)

    binv = (r_kk1 == c_kk1).astype(_F32) * jnp.ones((K, K, Q), _F32)
    aw = b
    for k in range(K):
        piv = aw[k:k + 1, k:k + 1, :]
        pr_a = aw[k:k + 1, :, :] / piv
        pr_i = binv[k:k + 1, :, :] / piv
        colf = aw[:, k:k + 1, :]
        isrow = r_kk1[:, 0:1, :] == k
        f = jnp.where(isrow, 0.0, colf)
        aw = jnp.where(isrow, pr_a, aw - f * pr_a)
        binv = jnp.where(isrow, pr_i, binv - f * pr_i)

    w = jnp.sum(kxz_t[:, None, :] * binv, axis=0)

    wg = ph[0] * w[0:1, :]
    for j in range(1, K):
        wg = wg + ph[j] * w[j:j + 1, :]

    swg = lax.dot_general(s_ref[:], wg, (((1,), (0,)), ((), ())),
                          preferred_element_type=_F32)
    quad_s = jnp.sum(wg * swg, axis=0, keepdims=True)
    mean = jnp.sum(wg * mu_ref[:], axis=0, keepdims=True)
    quad_k = (jnp.sum(w * kxz_t, axis=0, keepdims=True)
              - JITTER * jnp.sum(w * w, axis=0, keepdims=True))
    cov = VARIANCE - quad_k + quad_s
    mean_ref[:] = mean
    scale_ref[:] = jnp.sqrt(jnp.clip(cov, 0.05, None))


def kernel(X, Z, Lu_raw, mu):
    lu, l_mat, s_mat, zt, z2 = pl.pallas_call(
        _prep_body,
        out_shape=(
            jax.ShapeDtypeStruct((M, M), _F32),
            jax.ShapeDtypeStruct((M, M), _F32),
            jax.ShapeDtypeStruct((M, M), _F32),
            jax.ShapeDtypeStruct((DIM, M), _F32),
            jax.ShapeDtypeStruct((1, M), _F32),
        ),
        scratch_shapes=[pltpu.VMEM((M, M), _F32),
                        pltpu.VMEM((SB, M), _F32)],
    )(Z, Lu_raw)

    mu2 = mu.reshape(M, 1)
    grid = (N // Q,)
    mean, scale = pl.pallas_call(
        _main_body,
        grid=grid,
        in_specs=[
            pl.BlockSpec((Q, DIM), lambda i: (i, 0)),
            pl.BlockSpec((DIM, M), lambda i: (0, 0)),
            pl.BlockSpec((1, M), lambda i: (0, 0)),
            pl.BlockSpec((M, M), lambda i: (0, 0)),
            pl.BlockSpec((M, 1), lambda i: (0, 0)),
        ],
        out_specs=[
            pl.BlockSpec((1, Q), lambda i: (0, i)),
            pl.BlockSpec((1, Q), lambda i: (0, i)),
        ],
        out_shape=(
            jax.ShapeDtypeStruct((1, N), _F32),
            jax.ShapeDtypeStruct((1, N), _F32),
        ),
    )(X, zt, z2, s_mat, mu2)

    return mean, scale, mu, lu, l_mat


# speedup vs baseline: 1.0716x; 1.0716x over previous
import jax
import jax.numpy as jnp
from jax import lax
from jax.experimental import pallas as pl
from jax.experimental.pallas import tpu as pltpu

N = 16384
M = 256
K = 8
DIM = 16
JITTER = 1e-4
LENGTHSCALE = 1.0
VARIANCE = 1.0
Q = 256
NB = 128
SB = 32

_F32 = jnp.float32


def _prep_body(z_ref, lur_ref, lu_ref, l_ref, s_ref, zt_ref, z2_ref,
               lt_ref, spnl_ref):
    Z = z_ref[:]
    raw = lur_ref[:]
    r = lax.broadcasted_iota(jnp.int32, (M, M), 0)
    c = lax.broadcasted_iota(jnp.int32, (M, M), 1)
    eye = (r == c).astype(_F32)
    lu = jnp.where(c < r, raw, 0.0) + jnp.where(c == r, jnp.exp(raw), 0.0)
    lu_ref[:] = lu
    s_ref[:] = lax.dot_general(lu, lu, (((1,), (1,)), ((), ())),
                               preferred_element_type=_F32)
    zt_ref[:] = lax.dot_general(Z, eye, (((0,), (0,)), ((), ())),
                                preferred_element_type=_F32)
    z2c = jnp.sum(Z * Z, axis=1, keepdims=True)
    z2r = lax.dot_general(jnp.ones((1, DIM), _F32), Z * Z,
                          (((1,), (1,)), ((), ())),
                          preferred_element_type=_F32)
    z2_ref[:] = z2r
    zz = lax.dot_general(Z, Z, (((1,), (1,)), ((), ())),
                         preferred_element_type=_F32)
    d2 = jnp.maximum(z2c + z2r - 2.0 * zz, 0.0)
    kzz = VARIANCE * jnp.exp(-0.5 * d2 / (LENGTHSCALE ** 2))
    aw0 = kzz + JITTER * eye

    c_m = lax.broadcasted_iota(jnp.int32, (1, M), 1)
    lt_ref[:] = jnp.zeros((M, M), _F32)

    def make_body(s):
        def body(j, carry):
            rowj = spnl_ref[pl.ds(j, 1), :]
            jj = s + j
            ehot = (c_m == jj).astype(_F32)
            piv = jnp.sum(rowj * ehot, axis=1, keepdims=True)
            d = jnp.sqrt(piv)
            u = jnp.where(c_m > jj, rowj, 0.0) / d
            lt_ref[pl.ds(jj, 1), :] = u + d * ehot
            outer = lax.dot_general(u[:, s:s + SB], u,
                                    (((0,), (0,)), ((), ())),
                                    preferred_element_type=_F32)
            spnl_ref[:] = spnl_ref[:] - outer
            return carry
        return body

    for s in range(0, M, SB):
        blk = aw0[s:s + SB, :]
        if s:
            lts = lt_ref[:, s:s + SB]
            blk = blk - lax.dot_general(lts, lt_ref[:],
                                        (((0,), (0,)), ((), ())),
                                        preferred_element_type=_F32)
        spnl_ref[:] = blk
        lax.fori_loop(0, SB, make_body(s), 0)

    l_ref[:] = lax.dot_general(eye, lt_ref[:], (((1,), (1,)), ((), ())),
                               preferred_element_type=_F32)


def _main_body(x_ref, zt_ref, z2_ref, s_ref, mu_ref, mean_ref, scale_ref):
    x = x_ref[:]
    zt = zt_ref[:]
    z2 = z2_ref[:]
    x2 = jnp.sum(x * x, axis=1, keepdims=True)
    xz = lax.dot_general(x, zt, (((1,), (0,)), ((), ())),
                         preferred_element_type=_F32)
    d2 = jnp.maximum(x2 + z2 - 2.0 * xz, 0.0)
    lane_f = lax.broadcasted_iota(jnp.int32, (Q, M), 1).astype(_F32)
    idxs, vals = [], []
    d2w = d2
    for _ in range(K):
        mval = jnp.min(d2w, axis=1, keepdims=True)
        cand = jnp.where(d2w == mval, lane_f, _F32(1e9))
        am = jnp.min(cand, axis=1, keepdims=True)
        idxs.append(am)
        vals.append(mval)
        d2w = jnp.where(lane_f == am, _F32(3.4e38), d2w)
    idxf = jnp.concatenate(idxs, axis=1)
    d2k = jnp.concatenate(vals, axis=1)

    r_q = lax.broadcasted_iota(jnp.int32, (Q, Q), 0)
    c_q = lax.broadcasted_iota(jnp.int32, (Q, Q), 1)
    eye_q = (r_q == c_q).astype(_F32)
    idx_t = lax.dot_general(idxf, eye_q, (((0,), (0,)), ((), ())),
                            preferred_element_type=_F32)
    d2k_t = lax.dot_general(d2k, eye_q, (((0,), (0,)), ((), ())),
                            preferred_element_type=_F32)
    kxz_t = VARIANCE * jnp.exp(-0.5 * d2k_t / (LENGTHSCALE ** 2))

    r_mq = lax.broadcasted_iota(jnp.int32, (M, Q), 0).astype(_F32)
    ph = [(r_mq == idx_t[j:j + 1, :]).astype(_F32) for j in range(K)]
    zg = [lax.dot_general(zt, ph[j], (((1,), (0,)), ((), ())),
                          preferred_element_type=_F32) for j in range(K)]
    z2k = [jnp.sum(zg[j] * zg[j], axis=0, keepdims=True) for j in range(K)]

    gpair = {}
    for i in range(K):
        for j in range(i, K):
            gpair[(i, j)] = jnp.sum(zg[i] * zg[j], axis=0, keepdims=True)
    rows = []
    for i in range(K):
        for j in range(K):
            gij = gpair[(i, j)] if i <= j else gpair[(j, i)]
            rows.append(z2k[i] + z2k[j] - 2.0 * gij)
    d2z = jnp.maximum(jnp.concatenate(rows, axis=0), 0.0)
    ks = VARIANCE * jnp.exp(-0.5 * d2z / (LENGTHSCALE ** 2))
    r_kk1 = lax.broadcasted_iota(jnp.int32, (K, K, 1), 0)
    c_kk1 = lax.broadcasted_iota(jnp.int32, (K, K, 1), 1)
    b = ks.reshape(K, K, Q) + 2.0 * JITTER * (r_kk1 == c_kk1).astype(_F32)

    binv = (r_kk1 == c_kk1).astype(_F32) * jnp.ones((K, K, Q), _F32)
    aw = b
    for k in range(K):
        piv = aw[k:k + 1, k:k + 1, :]
        pr_a = aw[k:k + 1, :, :] / piv
        pr_i = binv[k:k + 1, :, :] / piv
        colf = aw[:, k:k + 1, :]
        isrow = r_kk1[:, 0:1, :] == k
        f = jnp.where(isrow, 0.0, colf)
        aw = jnp.where(isrow, pr_a, aw - f * pr_a)
        binv = jnp.where(isrow, pr_i, binv - f * pr_i)

    w = jnp.sum(kxz_t[:, None, :] * binv, axis=0)

    wg = ph[0] * w[0:1, :]
    for j in range(1, K):
        wg = wg + ph[j] * w[j:j + 1, :]

    swg = lax.dot_general(s_ref[:], wg, (((1,), (0,)), ((), ())),
                          preferred_element_type=_F32)
    quad_s = jnp.sum(wg * swg, axis=0, keepdims=True)
    mean = jnp.sum(wg * mu_ref[:], axis=0, keepdims=True)
    quad_k = (jnp.sum(w * kxz_t, axis=0, keepdims=True)
              - JITTER * jnp.sum(w * w, axis=0, keepdims=True))
    cov = VARIANCE - quad_k + quad_s
    mean_ref[:] = mean
    scale_ref[:] = jnp.sqrt(jnp.clip(cov, 0.05, None))


def kernel(X, Z, Lu_raw, mu):
    lu, l_mat, s_mat, zt, z2 = pl.pallas_call(
        _prep_body,
        out_shape=(
            jax.ShapeDtypeStruct((M, M), _F32),
            jax.ShapeDtypeStruct((M, M), _F32),
            jax.ShapeDtypeStruct((M, M), _F32),
            jax.ShapeDtypeStruct((DIM, M), _F32),
            jax.ShapeDtypeStruct((1, M), _F32),
        ),
        scratch_shapes=[pltpu.VMEM((M, M), _F32),
                        pltpu.VMEM((SB, M), _F32)],
    )(Z, Lu_raw)

    mu2 = mu.reshape(M, 1)
    grid = (N // Q,)
    mean, scale = pl.pallas_call(
        _main_body,
        grid=grid,
        in_specs=[
            pl.BlockSpec((Q, DIM), lambda i: (i, 0)),
            pl.BlockSpec((DIM, M), lambda i: (0, 0)),
            pl.BlockSpec((1, M), lambda i: (0, 0)),
            pl.BlockSpec((M, M), lambda i: (0, 0)),
            pl.BlockSpec((M, 1), lambda i: (0, 0)),
        ],
        out_specs=[
            pl.BlockSpec((1, Q), lambda i: (0, i)),
            pl.BlockSpec((1, Q), lambda i: (0, i)),
        ],
        out_shape=(
            jax.ShapeDtypeStruct((1, N), _F32),
            jax.ShapeDtypeStruct((1, N), _F32),
        ),
    )(X, zt, z2, s_mat, mu2)

    return mean, scale, mu, lu, l_mat
